# baseline (speedup 1.0000x reference)
import jax
import jax.numpy as jnp
from jax.experimental import pallas as pl
from jax.experimental.pallas import tpu as pltpu


def kernel(x, pos_emb):
    batch = x.shape[0]
    max_len, d_model = pos_emb.shape
    block_rows = 2048
    nblk = max_len // block_rows

    def body(p_ref, o_ref, buf, in_sem, out_sem):
        in_copies = [
            pltpu.make_async_copy(
                p_ref.at[pl.ds(i * block_rows, block_rows)],
                buf.at[i],
                in_sem.at[i],
            )
            for i in range(nblk)
        ]
        in_copies[0].start()
        out_copies = []
        for i in range(nblk):
            in_copies[i].wait()
            if i + 1 < nblk:
                in_copies[i + 1].start()
            for b in range(batch):
                c = pltpu.make_async_copy(
                    buf.at[i],
                    o_ref.at[b, pl.ds(i * block_rows, block_rows)],
                    out_sem.at[i, b],
                )
                c.start()
                out_copies.append(c)
        for c in out_copies:
            c.wait()

    return pl.pallas_call(
        body,
        in_specs=[pl.BlockSpec(memory_space=pl.ANY)],
        out_specs=pl.BlockSpec(memory_space=pl.ANY),
        out_shape=jax.ShapeDtypeStruct((batch, max_len, d_model), pos_emb.dtype),
        scratch_shapes=[
            pltpu.VMEM((nblk, block_rows, d_model), pos_emb.dtype),
            pltpu.SemaphoreType.DMA((nblk,)),
            pltpu.SemaphoreType.DMA((nblk, batch)),
        ],
    )(pos_emb)


# speedup vs baseline: 1.0404x; 1.0404x over previous
import jax
import jax.numpy as jnp
from jax.experimental import pallas as pl
from jax.experimental.pallas import tpu as pltpu


def kernel(x, pos_emb):
    batch = x.shape[0]
    max_len, d_model = pos_emb.shape
    block_rows = 4096
    nblk = max_len // block_rows

    def body(p_ref, o_ref, buf, in_sem, out_sem):
        in_copies = [
            pltpu.make_async_copy(
                p_ref.at[pl.ds(i * block_rows, block_rows)],
                buf.at[i],
                in_sem.at[i],
            )
            for i in range(nblk)
        ]
        for c in in_copies:
            c.start()
        out_copies = []
        for i in range(nblk):
            in_copies[i].wait()
            for b in range(batch):
                c = pltpu.make_async_copy(
                    buf.at[i],
                    o_ref.at[b, pl.ds(i * block_rows, block_rows)],
                    out_sem.at[i, b],
                )
                c.start()
                out_copies.append(c)
        for c in out_copies:
            c.wait()

    return pl.pallas_call(
        body,
        in_specs=[pl.BlockSpec(memory_space=pl.ANY)],
        out_specs=pl.BlockSpec(memory_space=pl.ANY),
        out_shape=jax.ShapeDtypeStruct((batch, max_len, d_model), pos_emb.dtype),
        scratch_shapes=[
            pltpu.VMEM((nblk, block_rows, d_model), pos_emb.dtype),
            pltpu.SemaphoreType.DMA((nblk,)),
            pltpu.SemaphoreType.DMA((nblk, batch)),
        ],
    )(pos_emb)
